# Initial kernel scaffold; baseline (speedup 1.0000x reference)
import functools

import jax
import jax.numpy as jnp
from jax import lax
from jax.experimental import pallas as pl
from jax.experimental.pallas import tpu as pltpu
from jax.experimental.pallas import tpu_sc as plsc

_NC = 2
_NS = 16
_NW = _NC * _NS
_L = 16

_F32 = jnp.float32


def _msg_mlp(x, W0, b0):
    N, D = x.shape
    BM = 1000

    def body(x_ref, w_ref, b_ref, o_ref):
        acc = lax.dot_general(
            x_ref[...], w_ref[...], (((1,), (0,)), ((), ())),
            precision=lax.Precision.HIGHEST, preferred_element_type=_F32)
        o_ref[...] = jnp.maximum(acc + b_ref[...], 0.0)

    return pl.pallas_call(
        body,
        grid=(N // BM,),
        in_specs=[
            pl.BlockSpec((BM, D), lambda i: (i, 0)),
            pl.BlockSpec((D, D), lambda i: (0, 0)),
            pl.BlockSpec((1, D), lambda i: (0, 0)),
        ],
        out_specs=pl.BlockSpec((BM, D), lambda i: (i, 0)),
        out_shape=jax.ShapeDtypeStruct((N, D), _F32),
    )(x, W0, b0.reshape(1, D))


def _sc_aggregate(m, src, dst):
    N, D = m.shape
    E = src.shape[0]
    C = 80
    EPW = E // _NW
    NCH = EPW // C
    RS = N // _NS
    ZR = 125

    src2 = src.reshape(E // C, C)
    dst2 = dst.reshape(E // C, C)

    mesh = plsc.VectorSubcoreMesh(core_axis_name="c", subcore_axis_name="s")

    @functools.partial(
        pl.kernel,
        out_type=[
            jax.ShapeDtypeStruct((_NC * N, D), _F32),
            jax.ShapeDtypeStruct((_NC * N, _L), _F32),
        ],
        mesh=mesh,
        scratch_types=[
            pltpu.VMEM((NCH, C), jnp.int32),
            pltpu.VMEM((NCH, C), jnp.int32),
            pltpu.VMEM((C, D), _F32),
            pltpu.VMEM((C, _L), _F32),
            pltpu.VMEM((ZR, D), _F32),
            pltpu.VMEM((ZR, _L), _F32),
            pltpu.VMEM_SHARED((N, D), _F32),
            pltpu.VMEM_SHARED((N, _L), _F32),
            pltpu.SemaphoreType.DMA,
        ],
    )
    def agg_kernel(m_hbm, src_hbm, dst_hbm, sum_hbm, cnt_hbm,
                   srcv, dstv, rows, ones, zb, zbc, acc_s, cnt_s, sem):
        cid = lax.axis_index("c")
        sid = lax.axis_index("s")
        wid = cid * _NS + sid

        @pl.loop(0, ZR)
        def _(r):
            zbc[r, pl.ds(0, _L)] = jnp.zeros((_L,), _F32)

            @pl.loop(0, D, step=_L)
            def _(c0):
                zb[r, pl.ds(c0, _L)] = jnp.zeros((_L,), _F32)

        @pl.loop(0, C)
        def _(r):
            ones[r, pl.ds(0, _L)] = jnp.ones((_L,), _F32)

        row0 = sid * RS
        for k in range(RS // ZR):
            pltpu.sync_copy(zb, acc_s.at[pl.ds(row0 + k * ZR, ZR)])
            pltpu.sync_copy(zbc, cnt_s.at[pl.ds(row0 + k * ZR, ZR)])
        plsc.subcore_barrier()

        pltpu.sync_copy(src_hbm.at[pl.ds(wid * NCH, NCH)], srcv)
        pltpu.sync_copy(dst_hbm.at[pl.ds(wid * NCH, NCH)], dstv)

        @pl.loop(0, NCH)
        def _(j):
            pltpu.async_copy(m_hbm.at[srcv.at[j]], rows, sem).wait()
            pltpu.sync_copy(rows, acc_s.at[dstv.at[j]], add=True)
            pltpu.sync_copy(ones, cnt_s.at[dstv.at[j]], add=True)
        plsc.subcore_barrier()

        for k in range(RS // ZR):
            r0 = row0 + k * ZR
            pltpu.sync_copy(acc_s.at[pl.ds(r0, ZR)],
                            sum_hbm.at[pl.ds(cid * N + r0, ZR)])
            pltpu.sync_copy(cnt_s.at[pl.ds(r0, ZR)],
                            cnt_hbm.at[pl.ds(cid * N + r0, ZR)])

    return agg_kernel(m, src2, dst2)


def _update(x, Wu, bu, sums, cnts, Wa, ba):
    N, D = x.shape
    K = Wa.shape[1]
    BM = 1000
    NB = N // BM

    def body(x_ref, wu_ref, bu_ref, a0_ref, a1_ref, c0_ref, c1_ref,
             wa_ref, ba_ref, o_ref):
        cnt = c0_ref[...][:, :1] + c1_ref[...][:, :1]
        agg = (a0_ref[...] + a1_ref[...]) / jnp.maximum(cnt, 1.0)
        h = lax.dot_general(
            x_ref[...], wu_ref[...], (((1,), (0,)), ((), ())),
            precision=lax.Precision.HIGHEST, preferred_element_type=_F32)
        h = jnp.maximum(h + bu_ref[...] + agg, 0.0)
        a = lax.dot_general(
            h, wa_ref[...], (((1,), (0,)), ((), ())),
            precision=lax.Precision.HIGHEST, preferred_element_type=_F32)
        a = a + ba_ref[...]
        mx = jnp.max(a, axis=1, keepdims=True)
        lse = jnp.log(jnp.sum(jnp.exp(a - mx), axis=1, keepdims=True)) + mx
        o_ref[...] = a - lse

    return pl.pallas_call(
        body,
        grid=(NB,),
        in_specs=[
            pl.BlockSpec((BM, D), lambda i: (i, 0)),
            pl.BlockSpec((D, D), lambda i: (0, 0)),
            pl.BlockSpec((1, D), lambda i: (0, 0)),
            pl.BlockSpec((BM, D), lambda i: (i, 0)),
            pl.BlockSpec((BM, D), lambda i: (i + NB, 0)),
            pl.BlockSpec((BM, _L), lambda i: (i, 0)),
            pl.BlockSpec((BM, _L), lambda i: (i + NB, 0)),
            pl.BlockSpec((D, K), lambda i: (0, 0)),
            pl.BlockSpec((1, K), lambda i: (0, 0)),
        ],
        out_specs=pl.BlockSpec((BM, K), lambda i: (i, 0)),
        out_shape=jax.ShapeDtypeStruct((N, K), _F32),
    )(x, Wu, bu.reshape(1, D), sums, sums, cnts, cnts, Wa, ba.reshape(1, K))


def kernel(x, edge_index, W0, b0, Wu, bu, Wa, ba):
    src = edge_index[0]
    dst = edge_index[1]
    m = _msg_mlp(x, W0, b0)
    sums, cnts = _sc_aggregate(m, src, dst)
    return _update(x, Wu, bu, sums, cnts, Wa, ba)


# speedup vs baseline: 1.0226x; 1.0226x over previous
import jax
import jax.numpy as jnp
from jax import lax
from jax.experimental import pallas as pl

_L = 16
_F32 = jnp.float32


def _msg_mlp(x, W0, b0):
    N, D = x.shape
    BM = 1000

    def body(x_ref, w_ref, b_ref, o_ref):
        acc = lax.dot_general(
            x_ref[...], w_ref[...], (((1,), (0,)), ((), ())),
            precision=lax.Precision.HIGHEST, preferred_element_type=_F32)
        o_ref[...] = jnp.maximum(acc + b_ref[...], 0.0)

    return pl.pallas_call(
        body,
        grid=(N // BM,),
        in_specs=[
            pl.BlockSpec((BM, D), lambda i: (i, 0)),
            pl.BlockSpec((D, D), lambda i: (0, 0)),
            pl.BlockSpec((1, D), lambda i: (0, 0)),
        ],
        out_specs=pl.BlockSpec((BM, D), lambda i: (i, 0)),
        out_shape=jax.ShapeDtypeStruct((N, D), _F32),
    )(x, W0, b0.reshape(1, D))


def _update(x, Wu, bu, agg, cnt, Wa, ba):
    N, D = x.shape
    K = Wa.shape[1]
    BM = 1000
    NB = N // BM

    def body(x_ref, wu_ref, bu_ref, a_ref, c_ref, wa_ref, ba_ref, o_ref):
        cnt_col = c_ref[...][:, :1]
        agg_mean = a_ref[...] / jnp.maximum(cnt_col, 1.0)
        h = lax.dot_general(
            x_ref[...], wu_ref[...], (((1,), (0,)), ((), ())),
            precision=lax.Precision.HIGHEST, preferred_element_type=_F32)
        h = jnp.maximum(h + bu_ref[...] + agg_mean, 0.0)
        a = lax.dot_general(
            h, wa_ref[...], (((1,), (0,)), ((), ())),
            precision=lax.Precision.HIGHEST, preferred_element_type=_F32)
        a = a + ba_ref[...]
        mx = jnp.max(a, axis=1, keepdims=True)
        lse = jnp.log(jnp.sum(jnp.exp(a - mx), axis=1, keepdims=True)) + mx
        o_ref[...] = a - lse

    return pl.pallas_call(
        body,
        grid=(NB,),
        in_specs=[
            pl.BlockSpec((BM, D), lambda i: (i, 0)),
            pl.BlockSpec((D, D), lambda i: (0, 0)),
            pl.BlockSpec((1, D), lambda i: (0, 0)),
            pl.BlockSpec((BM, D), lambda i: (i, 0)),
            pl.BlockSpec((BM, _L), lambda i: (i, 0)),
            pl.BlockSpec((D, K), lambda i: (0, 0)),
            pl.BlockSpec((1, K), lambda i: (0, 0)),
        ],
        out_specs=pl.BlockSpec((BM, K), lambda i: (i, 0)),
        out_shape=jax.ShapeDtypeStruct((N, K), _F32),
    )(x, Wu, bu.reshape(1, D), agg, cnt, Wa, ba.reshape(1, K))


def kernel(x, edge_index, W0, b0, Wu, bu, Wa, ba):
    N = x.shape[0]
    src = edge_index[0]
    dst = edge_index[1]
    m = _msg_mlp(x, W0, b0)
    msgs = jnp.take(m, src, axis=0)
    agg = jax.ops.segment_sum(msgs, dst, num_segments=N)
    cnt = jax.ops.segment_sum(jnp.ones((src.shape[0],), _F32), dst,
                              num_segments=N)
    cnt16 = jnp.broadcast_to(cnt[:, None], (N, _L))
    return _update(x, Wu, bu, agg, cnt16, Wa, ba)
